# baseline (speedup 1.0000x reference)
import functools

import jax
import jax.numpy as jnp
from jax import lax
from jax.experimental import pallas as pl
from jax.experimental.pallas import tpu as pltpu
from jax.experimental.pallas import tpu_sc as plsc

N = 10000
D = 128
E = 320000

NC = 2
NS = 16
NW = NC * NS
K = 128
EPT = 10240
EP = NW * EPT
CH = EPT // K
NACC = 10240
RPT = NACC // NS
RZ = 128


SB = 16


def _make_sc_agg(cha=CH, chb=CH):
    assert cha % SB == 0 and chb % SB == 0 and cha + chb == 2 * CH
    mesh = plsc.VectorSubcoreMesh(core_axis_name="c", subcore_axis_name="s")

    def body(feat, src2, dst2, out_p, src_l, dst_l, rows0, rows1, acc_sh,
             sem0, sem1):
        c = lax.axis_index("c")
        s = lax.axis_index("s")

        zero16 = jnp.zeros((16,), jnp.float32)
        nst = jnp.where(c == 0, cha // SB, chb // SB)
        base0 = jnp.where(c == 0, s * cha, NS * cha + s * chb)

        def zrow(i, carry):
            for j in range(D // 16):
                rows0[i, pl.ds(j * 16, 16)] = zero16
            return carry
        lax.fori_loop(0, K, zrow, 0)
        for t in range(RPT // K):
            pltpu.sync_copy(rows0, acc_sh.at[pl.ds(s * RPT + t * K, K)])
        plsc.subcore_barrier()

        def drain(buf, sem):
            pltpu.make_async_copy(feat.at[pl.ds(0, K)], buf, sem).wait()

        def stage(t, carry):
            base = pl.multiple_of(base0 + t * SB, 8)
            pltpu.sync_copy(src2.at[pl.ds(base, SB)], src_l)
            pltpu.sync_copy(dst2.at[pl.ds(base, SB)], dst_l)
            pltpu.async_copy(feat.at[src_l.at[0]], rows0, sem0)
            pltpu.async_copy(feat.at[src_l.at[1]], rows1, sem1)

            def step2(jj, carry2):
                j = jj * 2
                drain(rows0, sem0)
                pltpu.sync_copy(rows0, acc_sh.at[dst_l.at[j]], add=True)
                pltpu.async_copy(feat.at[src_l.at[lax.rem(j + 2, SB)]],
                                 rows0, sem0)
                drain(rows1, sem1)
                pltpu.sync_copy(rows1, acc_sh.at[dst_l.at[j + 1]], add=True)
                pltpu.async_copy(feat.at[src_l.at[lax.rem(j + 3, SB)]],
                                 rows1, sem1)
                return carry2
            lax.fori_loop(0, SB // 2, step2, 0)
            drain(rows0, sem0)
            drain(rows1, sem1)
            return carry
        lax.fori_loop(0, nst, stage, 0)

        plsc.subcore_barrier()
        row0 = pl.multiple_of(c * NACC + s * RPT, RPT)
        pltpu.sync_copy(acc_sh.at[pl.ds(s * RPT, RPT)],
                        out_p.at[pl.ds(row0, RPT)])

    return pl.kernel(
        body,
        mesh=mesh,
        out_type=[jax.ShapeDtypeStruct((NC * NACC, D), jnp.float32)],
        scratch_types=[
            pltpu.VMEM((SB, K), jnp.int32),
            pltpu.VMEM((SB, K), jnp.int32),
            pltpu.VMEM((K, D), jnp.float32),
            pltpu.VMEM((K, D), jnp.float32),
            pltpu.VMEM_SHARED((NACC, D), jnp.float32),
            pltpu.SemaphoreType.DMA,
            pltpu.SemaphoreType.DMA,
        ],
    )


DW = 128


def _make_sc_deg(dw=DW, scatter=True):
    mesh = plsc.VectorSubcoreMesh(core_axis_name="c", subcore_axis_name="s")

    def body(dst2, out_d, dst_l, ones_v, deg_sh, sem):
        c = lax.axis_index("c")
        s = lax.axis_index("s")

        zero16 = jnp.zeros((16,), jnp.float32)
        ones16 = jnp.ones((16,), jnp.float32)

        w = c * NS + s
        pltpu.sync_copy(dst2.at[pl.ds(pl.multiple_of(w * CH, CH), CH)], dst_l)

        def fill(v):
            def f(i, carry):
                for q in range(dw // 16):
                    ones_v[i, pl.ds(q * 16, 16)] = v
                return carry
            lax.fori_loop(0, K, f, 0)
        fill(zero16)
        for t in range(RPT // K):
            pltpu.sync_copy(ones_v, deg_sh.at[pl.ds(s * RPT + t * K, K)])
        fill(ones16)
        plsc.subcore_barrier()

        if scatter:
            def step(jj, carry):
                j = jj * 4
                for q in range(4):
                    pltpu.async_copy(ones_v, deg_sh.at[dst_l.at[j + q]],
                                     sem, add=True)
                for q in range(4):
                    pltpu.make_async_copy(
                        ones_v, deg_sh.at[pl.ds(0, K)], sem).wait()
                return carry
            lax.fori_loop(0, CH // 4, step, 0)

        plsc.subcore_barrier()
        row0 = pl.multiple_of(c * NACC + s * RPT, RPT)
        pltpu.sync_copy(deg_sh.at[pl.ds(s * RPT, RPT)],
                        out_d.at[pl.ds(row0, RPT)])

    return pl.kernel(
        body,
        mesh=mesh,
        out_type=[jax.ShapeDtypeStruct((NC * NACC, dw), jnp.float32)],
        scratch_types=[
            pltpu.VMEM((CH, K), jnp.int32),
            pltpu.VMEM((K, dw), jnp.float32),
            pltpu.VMEM_SHARED((NACC, dw), jnp.float32),
            pltpu.SemaphoreType.DMA,
        ],
    )


_sc_agg = _make_sc_agg(160, 0)
_sc_deg = _make_sc_deg()


def _combine_body(relu, p_ref, dg_ref, x_ref, wl_ref, wr_ref, b_ref, o_ref):
    deg = dg_ref[0, :, 0:1] + dg_ref[1, :, 0:1]
    inv = 1.0 / jnp.maximum(deg, 1.0)
    agg = (p_ref[0] + p_ref[1]) * inv
    r = jnp.dot(agg, wl_ref[...], preferred_element_type=jnp.float32)
    r = r + b_ref[...]
    r = r + jnp.dot(x_ref[...], wr_ref[...], preferred_element_type=jnp.float32)
    o_ref[...] = jnp.maximum(r, 0.0) if relu else r


def _combine(p, dg, xin, W_l, W_r, b, relu):
    B = 1000
    return pl.pallas_call(
        functools.partial(_combine_body, relu),
        grid=(N // B,),
        in_specs=[
            pl.BlockSpec((NC, B, D), lambda i: (0, i, 0)),
            pl.BlockSpec((NC, B, DW), lambda i: (0, i, 0)),
            pl.BlockSpec((B, D), lambda i: (i, 0)),
            pl.BlockSpec((D, D), lambda i: (0, 0)),
            pl.BlockSpec((D, D), lambda i: (0, 0)),
            pl.BlockSpec((1, D), lambda i: (0, 0)),
        ],
        out_specs=pl.BlockSpec((B, D), lambda i: (i, 0)),
        out_shape=jax.ShapeDtypeStruct((N, D), jnp.float32),
    )(p, dg, xin, W_l, W_r, b)


def kernel(x, edge_index, W1_l, W1_r, b1, W2_l, W2_r, b2):
    ei = edge_index.astype(jnp.int32)
    pad = EP - E
    srcp = jnp.concatenate([ei[0], jnp.zeros((pad,), jnp.int32)])
    dstp = jnp.concatenate([ei[1], jnp.full((pad,), N, jnp.int32)])
    src2 = srcp.reshape(NW * CH, K)
    dst2 = dstp.reshape(NW * CH, K)

    d1f, = _sc_deg(dst2)
    d1 = d1f.reshape(NC, NACC, DW)[:, :N]

    p1f, = _sc_agg(x, src2, dst2)
    p1 = p1f.reshape(NC, NACC, D)[:, :N]
    h = _combine(p1, d1, x, W1_l, W1_r, b1.reshape(1, D), relu=True)

    p2f, = _sc_agg(h, src2, dst2)
    p2 = p2f.reshape(NC, NACC, D)[:, :N]
    out = _combine(p2, d1, h, W2_l, W2_r, b2.reshape(1, D), relu=False)
    return out


# speedup vs baseline: 1.0848x; 1.0848x over previous
import functools

import jax
import jax.numpy as jnp
from jax import lax
from jax.experimental import pallas as pl
from jax.experimental.pallas import tpu as pltpu
from jax.experimental.pallas import tpu_sc as plsc

N = 10000
D = 128
E = 320000

NC = 2
NS = 16
NW = NC * NS
K = 128
EPT = 10240
EP = NW * EPT
CH = EPT // K
NACC = 10240
RPT = NACC // NS
RZ = 128


SB = 16


def _make_sc_agg(cha=CH, chb=CH):
    assert cha % SB == 0 and chb % SB == 0 and cha + chb == 2 * CH
    mesh = plsc.VectorSubcoreMesh(core_axis_name="c", subcore_axis_name="s")

    def body(feat, src2, dst2, out_p, src_l, dst_l, rows0, rows1, acc_sh,
             sem0, sem1):
        c = lax.axis_index("c")
        s = lax.axis_index("s")

        zero16 = jnp.zeros((16,), jnp.float32)
        nst = jnp.where(c == 0, cha // SB, chb // SB)
        base0 = jnp.where(c == 0, s * cha, NS * cha + s * chb)

        def zrow(i, carry):
            for j in range(D // 16):
                rows0[i, pl.ds(j * 16, 16)] = zero16
            return carry
        lax.fori_loop(0, K, zrow, 0)
        for t in range(RPT // K):
            pltpu.sync_copy(rows0, acc_sh.at[pl.ds(s * RPT + t * K, K)])
        plsc.subcore_barrier()

        def drain(buf, sem):
            pltpu.make_async_copy(feat.at[pl.ds(0, K)], buf, sem).wait()

        def stage(t, carry):
            base = pl.multiple_of(base0 + t * SB, 8)
            pltpu.sync_copy(src2.at[pl.ds(base, SB)], src_l)
            pltpu.sync_copy(dst2.at[pl.ds(base, SB)], dst_l)
            pltpu.async_copy(feat.at[src_l.at[0]], rows0, sem0)
            pltpu.async_copy(feat.at[src_l.at[1]], rows1, sem1)

            def step2(jj, carry2):
                j = jj * 2
                drain(rows0, sem0)
                pltpu.sync_copy(rows0, acc_sh.at[dst_l.at[j]], add=True)
                pltpu.async_copy(feat.at[src_l.at[lax.rem(j + 2, SB)]],
                                 rows0, sem0)
                drain(rows1, sem1)
                pltpu.sync_copy(rows1, acc_sh.at[dst_l.at[j + 1]], add=True)
                pltpu.async_copy(feat.at[src_l.at[lax.rem(j + 3, SB)]],
                                 rows1, sem1)
                return carry2
            lax.fori_loop(0, SB // 2, step2, 0)
            drain(rows0, sem0)
            drain(rows1, sem1)
            return carry
        lax.fori_loop(0, nst, stage, 0)

        plsc.subcore_barrier()
        row0 = pl.multiple_of(c * NACC + s * RPT, RPT)
        pltpu.sync_copy(acc_sh.at[pl.ds(s * RPT, RPT)],
                        out_p.at[pl.ds(row0, RPT)])

    return pl.kernel(
        body,
        mesh=mesh,
        out_type=[jax.ShapeDtypeStruct((NC * NACC, D), jnp.float32)],
        scratch_types=[
            pltpu.VMEM((SB, K), jnp.int32),
            pltpu.VMEM((SB, K), jnp.int32),
            pltpu.VMEM((K, D), jnp.float32),
            pltpu.VMEM((K, D), jnp.float32),
            pltpu.VMEM_SHARED((NACC, D), jnp.float32),
            pltpu.SemaphoreType.DMA,
            pltpu.SemaphoreType.DMA,
        ],
    )


DW = 128


def _make_sc_deg(dw=DW, scatter=True):
    mesh = plsc.VectorSubcoreMesh(core_axis_name="c", subcore_axis_name="s")

    def body(dst2, out_d, dst_l, ones_v, deg_sh, sem):
        c = lax.axis_index("c")
        s = lax.axis_index("s")

        zero16 = jnp.zeros((16,), jnp.float32)
        ones16 = jnp.ones((16,), jnp.float32)

        w = c * NS + s
        pltpu.sync_copy(dst2.at[pl.ds(pl.multiple_of(w * CH, CH), CH)], dst_l)

        def fill(v):
            def f(i, carry):
                for q in range(dw // 16):
                    ones_v[i, pl.ds(q * 16, 16)] = v
                return carry
            lax.fori_loop(0, K, f, 0)
        fill(zero16)
        for t in range(RPT // K):
            pltpu.sync_copy(ones_v, deg_sh.at[pl.ds(s * RPT + t * K, K)])
        fill(ones16)
        plsc.subcore_barrier()

        if scatter:
            def step(jj, carry):
                j = jj * 4
                for q in range(4):
                    pltpu.async_copy(ones_v, deg_sh.at[dst_l.at[j + q]],
                                     sem, add=True)
                for q in range(4):
                    pltpu.make_async_copy(
                        ones_v, deg_sh.at[pl.ds(0, K)], sem).wait()
                return carry
            lax.fori_loop(0, CH // 4, step, 0)

        plsc.subcore_barrier()
        row0 = pl.multiple_of(c * NACC + s * RPT, RPT)
        pltpu.sync_copy(deg_sh.at[pl.ds(s * RPT, RPT)],
                        out_d.at[pl.ds(row0, RPT)])

    return pl.kernel(
        body,
        mesh=mesh,
        out_type=[jax.ShapeDtypeStruct((NC * NACC, dw), jnp.float32)],
        scratch_types=[
            pltpu.VMEM((CH, K), jnp.int32),
            pltpu.VMEM((K, dw), jnp.float32),
            pltpu.VMEM_SHARED((NACC, dw), jnp.float32),
            pltpu.SemaphoreType.DMA,
        ],
    )


_sc_agg = _make_sc_agg()
_sc_deg = _make_sc_deg()


def _combine_body(relu, p_ref, dg_ref, x_ref, wl_ref, wr_ref, b_ref, o_ref):
    deg = dg_ref[0, :, 0:1] + dg_ref[1, :, 0:1]
    inv = 1.0 / jnp.maximum(deg, 1.0)
    agg = (p_ref[0] + p_ref[1]) * inv
    r = jnp.dot(agg, wl_ref[...], preferred_element_type=jnp.float32)
    r = r + b_ref[...]
    r = r + jnp.dot(x_ref[...], wr_ref[...], preferred_element_type=jnp.float32)
    o_ref[...] = jnp.maximum(r, 0.0) if relu else r


def _combine(p, dg, xin, W_l, W_r, b, relu):
    B = 1000
    return pl.pallas_call(
        functools.partial(_combine_body, relu),
        grid=(N // B,),
        in_specs=[
            pl.BlockSpec((NC, B, D), lambda i: (0, i, 0)),
            pl.BlockSpec((NC, B, DW), lambda i: (0, i, 0)),
            pl.BlockSpec((B, D), lambda i: (i, 0)),
            pl.BlockSpec((D, D), lambda i: (0, 0)),
            pl.BlockSpec((D, D), lambda i: (0, 0)),
            pl.BlockSpec((1, D), lambda i: (0, 0)),
        ],
        out_specs=pl.BlockSpec((B, D), lambda i: (i, 0)),
        out_shape=jax.ShapeDtypeStruct((N, D), jnp.float32),
    )(p, dg, xin, W_l, W_r, b)


def kernel(x, edge_index, W1_l, W1_r, b1, W2_l, W2_r, b2):
    ei = edge_index.astype(jnp.int32)
    pad = EP - E
    srcp = jnp.concatenate([ei[0], jnp.zeros((pad,), jnp.int32)])
    dstp = jnp.concatenate([ei[1], jnp.full((pad,), N, jnp.int32)])
    src2 = srcp.reshape(NW * CH, K)
    dst2 = dstp.reshape(NW * CH, K)

    d1f, = _sc_deg(dst2)
    d1 = d1f.reshape(NC, NACC, DW)[:, :N]

    p1f, = _sc_agg(x, src2, dst2)
    p1 = p1f.reshape(NC, NACC, D)[:, :N]
    h = _combine(p1, d1, x, W1_l, W1_r, b1.reshape(1, D), relu=True)

    p2f, = _sc_agg(h, src2, dst2)
    p2 = p2f.reshape(NC, NACC, D)[:, :N]
    out = _combine(p2, d1, h, W2_l, W2_r, b2.reshape(1, D), relu=False)
    return out


# speedup vs baseline: 1.2214x; 1.1260x over previous
import functools

import jax
import jax.numpy as jnp
from jax import lax
from jax.experimental import pallas as pl
from jax.experimental.pallas import tpu as pltpu
from jax.experimental.pallas import tpu_sc as plsc

N = 10000
D = 128
E = 320000

NC = 2
NS = 16
NW = NC * NS
K = 128
EPT = 10240
EP = NW * EPT
CH = EPT // K
NACC = 10240
RPT = NACC // NS
RZ = 128


SB = 32


def _make_sc_agg(cha=CH, chb=CH):
    assert cha % SB == 0 and chb % SB == 0 and cha + chb == 2 * CH
    mesh = plsc.VectorSubcoreMesh(core_axis_name="c", subcore_axis_name="s")

    def body(feat, src2, dst2, out_p, src_l, dst_l, rows0, rows1, acc_sh,
             sem0, sem1):
        c = lax.axis_index("c")
        s = lax.axis_index("s")

        zero16 = jnp.zeros((16,), jnp.float32)
        nst = jnp.where(c == 0, cha // SB, chb // SB)
        base0 = jnp.where(c == 0, s * cha, NS * cha + s * chb)

        def zrow(i, carry):
            for j in range(D // 16):
                rows0[i, pl.ds(j * 16, 16)] = zero16
            return carry
        lax.fori_loop(0, K, zrow, 0)
        for t in range(RPT // K):
            pltpu.sync_copy(rows0, acc_sh.at[pl.ds(s * RPT + t * K, K)])
        plsc.subcore_barrier()

        def drain(buf, sem):
            pltpu.make_async_copy(feat.at[pl.ds(0, K)], buf, sem).wait()

        def stage(t, carry):
            base = pl.multiple_of(base0 + t * SB, 8)
            pltpu.sync_copy(src2.at[pl.ds(base, SB)], src_l)
            pltpu.sync_copy(dst2.at[pl.ds(base, SB)], dst_l)
            pltpu.async_copy(feat.at[src_l.at[0]], rows0, sem0)
            pltpu.async_copy(feat.at[src_l.at[1]], rows1, sem1)

            def step2(jj, carry2):
                j = jj * 2
                drain(rows0, sem0)
                pltpu.sync_copy(rows0, acc_sh.at[dst_l.at[j]], add=True)
                pltpu.async_copy(feat.at[src_l.at[lax.rem(j + 2, SB)]],
                                 rows0, sem0)
                drain(rows1, sem1)
                pltpu.sync_copy(rows1, acc_sh.at[dst_l.at[j + 1]], add=True)
                pltpu.async_copy(feat.at[src_l.at[lax.rem(j + 3, SB)]],
                                 rows1, sem1)
                return carry2
            lax.fori_loop(0, SB // 2, step2, 0)
            drain(rows0, sem0)
            drain(rows1, sem1)
            return carry
        lax.fori_loop(0, nst, stage, 0)

        plsc.subcore_barrier()
        row0 = pl.multiple_of(c * NACC + s * RPT, RPT)
        pltpu.sync_copy(acc_sh.at[pl.ds(s * RPT, RPT)],
                        out_p.at[pl.ds(row0, RPT)])

    return pl.kernel(
        body,
        mesh=mesh,
        out_type=[jax.ShapeDtypeStruct((NC * NACC, D), jnp.float32)],
        scratch_types=[
            pltpu.VMEM((SB, K), jnp.int32),
            pltpu.VMEM((SB, K), jnp.int32),
            pltpu.VMEM((K, D), jnp.float32),
            pltpu.VMEM((K, D), jnp.float32),
            pltpu.VMEM_SHARED((NACC, D), jnp.float32),
            pltpu.SemaphoreType.DMA,
            pltpu.SemaphoreType.DMA,
        ],
    )


DW = 128


def _make_sc_deg(dw=DW, scatter=True):
    mesh = plsc.VectorSubcoreMesh(core_axis_name="c", subcore_axis_name="s")

    def body(dst2, out_d, dst_l, ones_v, deg_sh, sem):
        c = lax.axis_index("c")
        s = lax.axis_index("s")

        zero16 = jnp.zeros((16,), jnp.float32)
        ones16 = jnp.ones((16,), jnp.float32)

        w = c * NS + s
        pltpu.sync_copy(dst2.at[pl.ds(pl.multiple_of(w * CH, CH), CH)], dst_l)

        def fill(v):
            def f(i, carry):
                for q in range(dw // 16):
                    ones_v[i, pl.ds(q * 16, 16)] = v
                return carry
            lax.fori_loop(0, K, f, 0)
        fill(zero16)
        for t in range(RPT // K):
            pltpu.sync_copy(ones_v, deg_sh.at[pl.ds(s * RPT + t * K, K)])
        fill(ones16)
        plsc.subcore_barrier()

        if scatter:
            def step(jj, carry):
                j = jj * 4
                for q in range(4):
                    pltpu.async_copy(ones_v, deg_sh.at[dst_l.at[j + q]],
                                     sem, add=True)
                for q in range(4):
                    pltpu.make_async_copy(
                        ones_v, deg_sh.at[pl.ds(0, K)], sem).wait()
                return carry
            lax.fori_loop(0, CH // 4, step, 0)

        plsc.subcore_barrier()
        row0 = pl.multiple_of(c * NACC + s * RPT, RPT)
        pltpu.sync_copy(deg_sh.at[pl.ds(s * RPT, RPT)],
                        out_d.at[pl.ds(row0, RPT)])

    return pl.kernel(
        body,
        mesh=mesh,
        out_type=[jax.ShapeDtypeStruct((NC * NACC, dw), jnp.float32)],
        scratch_types=[
            pltpu.VMEM((CH, K), jnp.int32),
            pltpu.VMEM((K, dw), jnp.float32),
            pltpu.VMEM_SHARED((NACC, dw), jnp.float32),
            pltpu.SemaphoreType.DMA,
        ],
    )


_sc_agg = _make_sc_agg(128, 32)
_sc_deg = _make_sc_deg()


def _combine_body(relu, p_ref, dg_ref, x_ref, wl_ref, wr_ref, b_ref, o_ref):
    deg = dg_ref[0, :, 0:1] + dg_ref[1, :, 0:1]
    inv = 1.0 / jnp.maximum(deg, 1.0)
    agg = (p_ref[0] + p_ref[1]) * inv
    r = jnp.dot(agg, wl_ref[...], preferred_element_type=jnp.float32)
    r = r + b_ref[...]
    r = r + jnp.dot(x_ref[...], wr_ref[...], preferred_element_type=jnp.float32)
    o_ref[...] = jnp.maximum(r, 0.0) if relu else r


def _combine(p, dg, xin, W_l, W_r, b, relu):
    B = 1000
    return pl.pallas_call(
        functools.partial(_combine_body, relu),
        grid=(N // B,),
        in_specs=[
            pl.BlockSpec((NC, B, D), lambda i: (0, i, 0)),
            pl.BlockSpec((NC, B, DW), lambda i: (0, i, 0)),
            pl.BlockSpec((B, D), lambda i: (i, 0)),
            pl.BlockSpec((D, D), lambda i: (0, 0)),
            pl.BlockSpec((D, D), lambda i: (0, 0)),
            pl.BlockSpec((1, D), lambda i: (0, 0)),
        ],
        out_specs=pl.BlockSpec((B, D), lambda i: (i, 0)),
        out_shape=jax.ShapeDtypeStruct((N, D), jnp.float32),
    )(p, dg, xin, W_l, W_r, b)


def kernel(x, edge_index, W1_l, W1_r, b1, W2_l, W2_r, b2):
    ei = edge_index.astype(jnp.int32)
    pad = EP - E
    srcp = jnp.concatenate([ei[0], jnp.zeros((pad,), jnp.int32)])
    dstp = jnp.concatenate([ei[1], jnp.full((pad,), N, jnp.int32)])
    src2 = srcp.reshape(NW * CH, K)
    dst2 = dstp.reshape(NW * CH, K)

    d1f, = _sc_deg(dst2)
    d1 = d1f.reshape(NC, NACC, DW)[:, :N]

    p1f, = _sc_agg(x, src2, dst2)
    p1 = p1f.reshape(NC, NACC, D)[:, :N]
    h = _combine(p1, d1, x, W1_l, W1_r, b1.reshape(1, D), relu=True)

    p2f, = _sc_agg(h, src2, dst2)
    p2 = p2f.reshape(NC, NACC, D)[:, :N]
    out = _combine(p2, d1, h, W2_l, W2_r, b2.reshape(1, D), relu=False)
    return out
